# Initial kernel scaffold; baseline (speedup 1.0000x reference)
import jax
import jax.numpy as jnp
from jax.experimental import pallas as pl


def kernel(sc, oc, pos, device, W1, b1, W2, b2, Wc0, bc0, Wc1, bc1, Wt, bt, Ws1, bs1, Ws2, bs2, Ws3, bs3, Ws4, bs4, Wf1, bf1, Wf2, bf2, Wf3, bf3):
    raise NotImplementedError("write your pallas kernel here")


# speedup vs baseline: 6.4133x; 6.4133x over previous
import functools

import jax
import jax.numpy as jnp
from jax import lax
from jax.experimental import pallas as pl
from jax.experimental.pallas import tpu as pltpu
from jax.experimental.pallas import tpu_sc as plsc

_NUM = 16
_P = 2048
_CS = 20
_NV = _CS * _CS
_F = 256
_HALF = 128
_CH = 512

_f32 = jnp.float32


def _point_kernel(sc_ref, cut_ref, w1_ref, b1_ref, w2_ref, b2_ref,
                  h_ref, seg_ref):
    v = sc_ref[0]
    x = v[:, 0:1]
    y = v[:, 1:2]
    cut = cut_ref[0:1, :]
    centers = (cut[:, 0:_CS] + cut[:, 1:_CS + 1]) * 0.5

    ix = jnp.sum((x > cut).astype(jnp.int32), axis=1, keepdims=True) - 1
    iy = jnp.sum((y > cut).astype(jnp.int32), axis=1, keepdims=True) - 1
    valid = (ix >= 0) & (ix < _CS) & (iy >= 0) & (iy < _CS)
    ixc = jnp.clip(ix, 0, _CS - 1)
    iyc = jnp.clip(iy, 0, _CS - 1)

    lane = lax.broadcasted_iota(jnp.int32, (_P, _CS), 1)
    ctrx = jnp.sum(jnp.where(lane == ixc, centers, 0.0), axis=1, keepdims=True)
    ctry = jnp.sum(jnp.where(lane == iyc, centers, 0.0), axis=1, keepdims=True)

    px = x - ctrx
    py = y - ctry
    h1 = jnp.maximum(px * w1_ref[0:1, :] + py * w1_ref[1:2, :] + b1_ref[0:1, :], 0.0)
    h = jnp.dot(h1, w2_ref[...], preferred_element_type=_f32, precision=lax.Precision.HIGHEST) + b2_ref[0:1, :]
    h_ref[0] = jnp.maximum(h, 0.0)
    seg_ref[0] = jnp.where(valid, ixc * _CS + iyc, _NV)


def _point_stage(sc, cut, W1, b1, W2, b2):
    return pl.pallas_call(
        _point_kernel,
        grid=(_NUM,),
        in_specs=[
            pl.BlockSpec((1, _P, 2), lambda n: (n, 0, 0)),
            pl.BlockSpec((1, _CS + 1), lambda n: (0, 0)),
            pl.BlockSpec((2, 128), lambda n: (0, 0)),
            pl.BlockSpec((1, 128), lambda n: (0, 0)),
            pl.BlockSpec((128, _F), lambda n: (0, 0)),
            pl.BlockSpec((1, _F), lambda n: (0, 0)),
        ],
        out_specs=[
            pl.BlockSpec((1, _P, _F), lambda n: (n, 0, 0)),
            pl.BlockSpec((1, _P, 1), lambda n: (n, 0, 0)),
        ],
        out_shape=[
            jax.ShapeDtypeStruct((_NUM, _P, _F), _f32),
            jax.ShapeDtypeStruct((_NUM, _P, 1), jnp.int32),
        ],
    )(sc, cut, W1, b1.reshape(1, -1), W2, b2.reshape(1, -1))


def _scatter_max_body(h_hbm, seg_hbm, vox_hbm, acc, hbuf, segbuf):
    scene = lax.axis_index("s")
    half = lax.axis_index("c")

    @pl.loop(0, _NV + 1)
    def _zero(r):
        for j in range(_HALF // 16):
            acc[r, pl.ds(j * 16, 16)] = jnp.zeros((16,), _f32)

    pltpu.sync_copy(seg_hbm.at[scene], segbuf.at[pl.ds(0, _P)])

    @pl.loop(0, _P // _CH)
    def _chunk(c):
        pltpu.sync_copy(
            h_hbm.at[scene, pl.ds(c * _CH, _CH), pl.ds(half * _HALF, _HALF)],
            hbuf)

        @pl.loop(0, _CH)
        def _point(p):
            s = segbuf[pl.ds(c * _CH + p, 16)][0]
            for j in range(_HALF // 16):
                sl = pl.ds(j * 16, 16)
                acc[s, sl] = jnp.maximum(acc[s, sl], hbuf[p, sl])

    pltpu.sync_copy(acc.at[pl.ds(0, _NV), :],
                    vox_hbm.at[scene, :, pl.ds(half * _HALF, _HALF)])


def _scatter_stage(h, seg):
    mesh = plsc.VectorSubcoreMesh(core_axis_name="c", subcore_axis_name="s")
    run = pl.kernel(
        _scatter_max_body,
        out_type=jax.ShapeDtypeStruct((_NUM, _NV, _F), _f32),
        mesh=mesh,
        scratch_types=[
            pltpu.VMEM((_NV + 1, _HALF), _f32),
            pltpu.VMEM((_CH, _HALF), _f32),
            pltpu.VMEM((_P + 16,), jnp.int32),
        ],
    )
    return run(h, seg)


def _conv_pool_kernel(hw, hwp, rows, x_ref, w_ref, b_ref, o_ref, pad_ref):
    c_in = x_ref.shape[2]
    c_out = w_ref.shape[1]
    pad_ref[...] = jnp.zeros_like(pad_ref)
    xv = x_ref[0]
    for r in range(hw):
        pad_ref[hwp * (r + 1) + 1:hwp * (r + 1) + 1 + hw, :] = \
            xv[hw * r:hw * r + hw, :]
    acc = jnp.zeros((rows, c_out), _f32)
    for d in range(9):
        off = hwp * (d // 3) + (d % 3)
        acc = acc + jnp.dot(pad_ref[off:off + rows, :],
                            w_ref[d * c_in:(d + 1) * c_in, :],
                            preferred_element_type=_f32, precision=lax.Precision.HIGHEST)
    act = jnp.maximum(acc + b_ref[...].reshape(1, c_out), 0.0)
    gr = act[:hw * hwp].reshape(hw, hwp, c_out)[:, :hw, :]
    p = gr.reshape(hw // 2, 2, hw // 2, 2, c_out)
    pooled = jnp.max(jnp.max(p, axis=3), axis=1)
    o_ref[0] = pooled.reshape((hw // 2) * (hw // 2), c_out)


def _conv0_stage(vox, w, b):
    kfn = functools.partial(_conv_pool_kernel, 20, 22, 448)
    return pl.pallas_call(
        kfn,
        grid=(_NUM,),
        in_specs=[
            pl.BlockSpec((1, _NV, _F), lambda n: (n, 0, 0)),
            pl.BlockSpec((9 * _F, 512), lambda n: (0, 0)),
            pl.BlockSpec((1, 512), lambda n: (0, 0)),
        ],
        out_specs=pl.BlockSpec((1, 100, 512), lambda n: (n, 0, 0)),
        out_shape=jax.ShapeDtypeStruct((_NUM, 100, 512), _f32),
        scratch_shapes=[pltpu.VMEM((22 * 22 + 60, _F), _f32)],
    )(vox, w, b.reshape(1, -1))


def _conv1_stage(l1, w, b):
    kfn = functools.partial(_conv_pool_kernel, 10, 12, 128)
    return pl.pallas_call(
        kfn,
        grid=(_NUM, 2),
        in_specs=[
            pl.BlockSpec((1, 100, 512), lambda n, o: (n, 0, 0)),
            pl.BlockSpec((9 * 512, 512), lambda n, o: (0, o)),
            pl.BlockSpec((1, 1, 512), lambda n, o: (o, 0, 0)),
        ],
        out_specs=pl.BlockSpec((1, 25, 512), lambda n, o: (n, 0, o)),
        out_shape=jax.ShapeDtypeStruct((_NUM, 25, 1024), _f32),
        scratch_shapes=[pltpu.VMEM((12 * 12 + 40, 512), _f32)],
    )(l1, w, b.reshape(2, 1, 512))


def _deconv_kernel(x_ref, w_ref, b_ref, o_ref):
    t = jnp.dot(x_ref[0], w_ref[...], preferred_element_type=_f32, precision=lax.Precision.HIGHEST)
    t = t.reshape(5, 5, 2, 2, 512)
    t = jnp.transpose(t, (0, 2, 1, 3, 4))
    o_ref[0] = t.reshape(100, 512) + b_ref[0:1, :]


def _deconv_stage(l2, wt, bt):
    return pl.pallas_call(
        _deconv_kernel,
        grid=(_NUM,),
        in_specs=[
            pl.BlockSpec((1, 25, 1024), lambda n: (n, 0, 0)),
            pl.BlockSpec((1024, 2048), lambda n: (0, 0)),
            pl.BlockSpec((1, 512), lambda n: (0, 0)),
        ],
        out_specs=pl.BlockSpec((1, 100, 512), lambda n: (n, 0, 0)),
        out_shape=jax.ShapeDtypeStruct((_NUM, 100, 512), _f32),
    )(l2, wt, bt.reshape(1, -1))


def _obj_kernel(oc_ref, w1_ref, b1_ref, w2_ref, b2_ref, w3_ref, b3_ref,
                w4_ref, b4_ref, o_ref):
    v = oc_ref[0]
    occ = v - jnp.mean(v, axis=0, keepdims=True)
    g = jnp.maximum(occ[:, 0:1] * w1_ref[0:1, :] + occ[:, 1:2] * w1_ref[1:2, :]
                    + b1_ref[0:1, :], 0.0)
    g = jnp.maximum(jnp.dot(g, w2_ref[...], preferred_element_type=_f32, precision=lax.Precision.HIGHEST)
                    + b2_ref[0:1, :], 0.0)
    g = jnp.maximum(jnp.dot(g, w3_ref[...], preferred_element_type=_f32, precision=lax.Precision.HIGHEST)
                    + b3_ref[0:1, :], 0.0)
    g = jnp.maximum(jnp.dot(g, w4_ref[...], preferred_element_type=_f32, precision=lax.Precision.HIGHEST)
                    + b4_ref[0:1, :], 0.0)
    o_ref[0] = jnp.max(g, axis=0, keepdims=True)


def _obj_stage(oc, Ws1, bs1, Ws2, bs2, Ws3, bs3, Ws4, bs4):
    q = oc.shape[1]
    return pl.pallas_call(
        _obj_kernel,
        grid=(_NUM,),
        in_specs=[
            pl.BlockSpec((1, q, 2), lambda n: (n, 0, 0)),
            pl.BlockSpec((2, 64), lambda n: (0, 0)),
            pl.BlockSpec((1, 64), lambda n: (0, 0)),
            pl.BlockSpec((64, 128), lambda n: (0, 0)),
            pl.BlockSpec((1, 128), lambda n: (0, 0)),
            pl.BlockSpec((128, 256), lambda n: (0, 0)),
            pl.BlockSpec((1, 256), lambda n: (0, 0)),
            pl.BlockSpec((256, 512), lambda n: (0, 0)),
            pl.BlockSpec((1, 512), lambda n: (0, 0)),
        ],
        out_specs=pl.BlockSpec((1, 1, 512), lambda n: (n, 0, 0)),
        out_shape=jax.ShapeDtypeStruct((_NUM, 1, 512), _f32),
    )(oc, Ws1, bs1.reshape(1, -1), Ws2, bs2.reshape(1, -1),
      Ws3, bs3.reshape(1, -1), Ws4, bs4.reshape(1, -1))


def _cls_kernel(l1_ref, l3_ref, obj_ref, pos_ref, ngrid_ref,
                a1_ref, a2_ref, bmat_ref, cmat_ref, bf1_ref,
                w2_ref, b2_ref, w3_ref, b3_ref, o_ref):
    rel = pos_ref[0] - ngrid_ref[...]
    y = jnp.dot(l1_ref[0], a1_ref[...], preferred_element_type=_f32, precision=lax.Precision.HIGHEST)
    y = y + jnp.dot(l3_ref[0], a2_ref[...], preferred_element_type=_f32, precision=lax.Precision.HIGHEST)
    y = y + jnp.dot(obj_ref[0], bmat_ref[...], preferred_element_type=_f32, precision=lax.Precision.HIGHEST)
    y = y + rel[:, 0:1] * cmat_ref[0:1, :] + rel[:, 1:2] * cmat_ref[1:2, :]
    y = jnp.maximum(y + bf1_ref[0:1, :], 0.0)
    y = jnp.maximum(jnp.dot(y, w2_ref[...], preferred_element_type=_f32, precision=lax.Precision.HIGHEST)
                    + b2_ref[0:1, :], 0.0)
    o_ref[0] = jnp.dot(y, w3_ref[...], preferred_element_type=_f32, precision=lax.Precision.HIGHEST) \
        + b3_ref[0:1, :]


def _cls_stage(l1, l3, obj, pos, ngrid, Wf1, bf1, Wf2, bf2, Wf3, bf3):
    return pl.pallas_call(
        _cls_kernel,
        grid=(_NUM,),
        in_specs=[
            pl.BlockSpec((1, 100, 512), lambda n: (n, 0, 0)),
            pl.BlockSpec((1, 100, 512), lambda n: (n, 0, 0)),
            pl.BlockSpec((1, 1, 512), lambda n: (n, 0, 0)),
            pl.BlockSpec((1, 1, 2), lambda n: (n, 0, 0)),
            pl.BlockSpec((100, 2), lambda n: (0, 0)),
            pl.BlockSpec((512, 1024), lambda n: (0, 0)),
            pl.BlockSpec((512, 1024), lambda n: (0, 0)),
            pl.BlockSpec((512, 1024), lambda n: (0, 0)),
            pl.BlockSpec((2, 1024), lambda n: (0, 0)),
            pl.BlockSpec((1, 1024), lambda n: (0, 0)),
            pl.BlockSpec((1024, 256), lambda n: (0, 0)),
            pl.BlockSpec((1, 256), lambda n: (0, 0)),
            pl.BlockSpec((256, 1), lambda n: (0, 0)),
            pl.BlockSpec((1, 1), lambda n: (0, 0)),
        ],
        out_specs=pl.BlockSpec((1, 100, 1), lambda n: (n, 0, 0)),
        out_shape=jax.ShapeDtypeStruct((_NUM, 100, 1), _f32),
    )(l1, l3, obj, pos.reshape(_NUM, 1, 2), ngrid,
      Wf1[:512], Wf1[512:1024], Wf1[1024:1536], Wf1[1536:1538],
      bf1.reshape(1, -1), Wf2, bf2.reshape(1, -1), Wf3, bf3.reshape(1, -1))


def kernel(sc, oc, pos, device, W1, b1, W2, b2, Wc0, bc0, Wc1, bc1, Wt, bt,
           Ws1, bs1, Ws2, bs2, Ws3, bs3, Ws4, bs4,
           Wf1, bf1, Wf2, bf2, Wf3, bf3):
    cut = jnp.linspace(-0.5, 0.5, _CS + 1, dtype=_f32).reshape(1, _CS + 1)
    ncut = jnp.linspace(-0.5, 0.5, 11, dtype=_f32)
    nctr = (ncut[:-1] + ncut[1:]) * 0.5
    ngrid = jnp.stack(jnp.meshgrid(nctr, nctr, indexing='ij'), axis=-1)
    ngrid = ngrid.reshape(100, 2)

    h, seg = _point_stage(sc, cut, W1, b1, W2, b2)
    vox = _scatter_stage(h, seg.reshape(_NUM, _P))

    wc0 = Wc0.transpose(2, 3, 1, 0).reshape(9 * _F, 512)
    l1 = _conv0_stage(vox, wc0, bc0)
    wc1 = Wc1.transpose(2, 3, 1, 0).reshape(9 * 512, 1024)
    l2 = _conv1_stage(l1, wc1, bc1)
    wt = Wt.transpose(0, 2, 3, 1).reshape(1024, 4 * 512)
    l3 = _deconv_stage(l2, wt, bt)

    obj = _obj_stage(oc, Ws1, bs1, Ws2, bs2, Ws3, bs3, Ws4, bs4)
    y = _cls_stage(l1, l3, obj, pos, ngrid, Wf1, bf1, Wf2, bf2, Wf3, bf3)
    return y


# speedup vs baseline: 11.3998x; 1.7775x over previous
import functools

import jax
import jax.numpy as jnp
from jax import lax
from jax.experimental import pallas as pl
from jax.experimental.pallas import tpu as pltpu
from jax.experimental.pallas import tpu_sc as plsc

_NUM = 16
_P = 2048
_CS = 20
_NV = _CS * _CS
_F = 256
_HALF = 128
_CH = 512

_f32 = jnp.float32


def _point_kernel(sc_ref, cut_ref, w1_ref, b1_ref, w2_ref, b2_ref,
                  h_ref, seg_ref):
    v = sc_ref[0]
    x = v[:, 0:1]
    y = v[:, 1:2]
    cut = cut_ref[0:1, :]
    centers = (cut[:, 0:_CS] + cut[:, 1:_CS + 1]) * 0.5

    ix = jnp.sum((x > cut).astype(jnp.int32), axis=1, keepdims=True) - 1
    iy = jnp.sum((y > cut).astype(jnp.int32), axis=1, keepdims=True) - 1
    valid = (ix >= 0) & (ix < _CS) & (iy >= 0) & (iy < _CS)
    ixc = jnp.clip(ix, 0, _CS - 1)
    iyc = jnp.clip(iy, 0, _CS - 1)

    lane = lax.broadcasted_iota(jnp.int32, (_P, _CS), 1)
    ctrx = jnp.sum(jnp.where(lane == ixc, centers, 0.0), axis=1, keepdims=True)
    ctry = jnp.sum(jnp.where(lane == iyc, centers, 0.0), axis=1, keepdims=True)

    px = x - ctrx
    py = y - ctry
    h1 = jnp.maximum(px * w1_ref[0:1, :] + py * w1_ref[1:2, :] + b1_ref[0:1, :], 0.0)
    h = jnp.dot(h1, w2_ref[...], preferred_element_type=_f32) + b2_ref[0:1, :]
    h_ref[0] = jnp.maximum(h, 0.0)
    seg_ref[0] = jnp.where(valid, ixc * _CS + iyc, _NV)


def _point_stage(sc, cut, W1, b1, W2, b2):
    return pl.pallas_call(
        _point_kernel,
        grid=(_NUM,),
        in_specs=[
            pl.BlockSpec((1, _P, 2), lambda n: (n, 0, 0)),
            pl.BlockSpec((1, _CS + 1), lambda n: (0, 0)),
            pl.BlockSpec((2, 128), lambda n: (0, 0)),
            pl.BlockSpec((1, 128), lambda n: (0, 0)),
            pl.BlockSpec((128, _F), lambda n: (0, 0)),
            pl.BlockSpec((1, _F), lambda n: (0, 0)),
        ],
        out_specs=[
            pl.BlockSpec((1, _P, _F), lambda n: (n, 0, 0)),
            pl.BlockSpec((1, _P, 1), lambda n: (n, 0, 0)),
        ],
        out_shape=[
            jax.ShapeDtypeStruct((_NUM, _P, _F), _f32),
            jax.ShapeDtypeStruct((_NUM, _P, 1), jnp.int32),
        ],
    )(sc, cut, W1, b1.reshape(1, -1), W2, b2.reshape(1, -1))


def _scatter_max_body(h_hbm, seg_hbm, vox_hbm, acc, hbuf, segbuf):
    scene = lax.axis_index("s")
    half = lax.axis_index("c")

    @pl.loop(0, _NV + 1)
    def _zero(r):
        for j in range(_HALF // 16):
            acc[r, pl.ds(j * 16, 16)] = jnp.zeros((16,), _f32)

    pltpu.sync_copy(seg_hbm.at[scene], segbuf.at[pl.ds(0, _P)])

    @pl.loop(0, _P // _CH)
    def _chunk(c):
        pltpu.sync_copy(
            h_hbm.at[scene, pl.ds(c * _CH, _CH), pl.ds(half * _HALF, _HALF)],
            hbuf)

        @pl.loop(0, _CH)
        def _point(p):
            s = segbuf[pl.ds(c * _CH + p, 16)][0]
            for j in range(_HALF // 16):
                sl = pl.ds(j * 16, 16)
                acc[s, sl] = jnp.maximum(acc[s, sl], hbuf[p, sl])

    pltpu.sync_copy(acc.at[pl.ds(0, _NV), :],
                    vox_hbm.at[scene, :, pl.ds(half * _HALF, _HALF)])


def _scatter_stage(h, seg):
    mesh = plsc.VectorSubcoreMesh(core_axis_name="c", subcore_axis_name="s")
    run = pl.kernel(
        _scatter_max_body,
        out_type=jax.ShapeDtypeStruct((_NUM, _NV, _F), _f32),
        mesh=mesh,
        scratch_types=[
            pltpu.VMEM((_NV + 1, _HALF), _f32),
            pltpu.VMEM((_CH, _HALF), _f32),
            pltpu.VMEM((_P + 16,), jnp.int32),
        ],
    )
    return run(h, seg)


def _conv_pool_kernel(hw, hwp, rows, x_ref, w_ref, b_ref, o_ref, pad_ref):
    c_in = x_ref.shape[2]
    c_out = w_ref.shape[1]
    pad_ref[...] = jnp.zeros_like(pad_ref)
    xv = x_ref[0]
    for r in range(hw):
        pad_ref[hwp * (r + 1) + 1:hwp * (r + 1) + 1 + hw, :] = \
            xv[hw * r:hw * r + hw, :]
    acc = jnp.zeros((rows, c_out), _f32)
    for d in range(9):
        off = hwp * (d // 3) + (d % 3)
        acc = acc + jnp.dot(pad_ref[off:off + rows, :],
                            w_ref[d * c_in:(d + 1) * c_in, :],
                            preferred_element_type=_f32)
    act = jnp.maximum(acc + b_ref[...].reshape(1, c_out), 0.0)
    gr = act[:hw * hwp].reshape(hw, hwp, c_out)[:, :hw, :]
    p = gr.reshape(hw // 2, 2, hw // 2, 2, c_out)
    pooled = jnp.max(jnp.max(p, axis=3), axis=1)
    o_ref[0] = pooled.reshape((hw // 2) * (hw // 2), c_out)


def _conv0_stage(vox, w, b):
    kfn = functools.partial(_conv_pool_kernel, 20, 22, 448)
    return pl.pallas_call(
        kfn,
        grid=(_NUM,),
        in_specs=[
            pl.BlockSpec((1, _NV, _F), lambda n: (n, 0, 0)),
            pl.BlockSpec((9 * _F, 512), lambda n: (0, 0)),
            pl.BlockSpec((1, 512), lambda n: (0, 0)),
        ],
        out_specs=pl.BlockSpec((1, 100, 512), lambda n: (n, 0, 0)),
        out_shape=jax.ShapeDtypeStruct((_NUM, 100, 512), _f32),
        scratch_shapes=[pltpu.VMEM((22 * 22 + 60, _F), _f32)],
    )(vox, w, b.reshape(1, -1))


def _conv1_stage(l1, w, b):
    kfn = functools.partial(_conv_pool_kernel, 10, 12, 128)
    return pl.pallas_call(
        kfn,
        grid=(_NUM, 2),
        in_specs=[
            pl.BlockSpec((1, 100, 512), lambda n, o: (n, 0, 0)),
            pl.BlockSpec((9 * 512, 512), lambda n, o: (0, o)),
            pl.BlockSpec((1, 1, 512), lambda n, o: (o, 0, 0)),
        ],
        out_specs=pl.BlockSpec((1, 25, 512), lambda n, o: (n, 0, o)),
        out_shape=jax.ShapeDtypeStruct((_NUM, 25, 1024), _f32),
        scratch_shapes=[pltpu.VMEM((12 * 12 + 40, 512), _f32)],
    )(l1, w, b.reshape(2, 1, 512))


def _deconv_kernel(x_ref, w_ref, b_ref, o_ref):
    t = jnp.dot(x_ref[0], w_ref[...], preferred_element_type=_f32)
    t = t.reshape(5, 5, 2, 2, 512)
    t = jnp.transpose(t, (0, 2, 1, 3, 4))
    o_ref[0] = t.reshape(100, 512) + b_ref[0:1, :]


def _deconv_stage(l2, wt, bt):
    return pl.pallas_call(
        _deconv_kernel,
        grid=(_NUM,),
        in_specs=[
            pl.BlockSpec((1, 25, 1024), lambda n: (n, 0, 0)),
            pl.BlockSpec((1024, 2048), lambda n: (0, 0)),
            pl.BlockSpec((1, 512), lambda n: (0, 0)),
        ],
        out_specs=pl.BlockSpec((1, 100, 512), lambda n: (n, 0, 0)),
        out_shape=jax.ShapeDtypeStruct((_NUM, 100, 512), _f32),
    )(l2, wt, bt.reshape(1, -1))


def _obj_kernel(oc_ref, w1_ref, b1_ref, w2_ref, b2_ref, w3_ref, b3_ref,
                w4_ref, b4_ref, o_ref):
    v = oc_ref[0]
    occ = v - jnp.mean(v, axis=0, keepdims=True)
    g = jnp.maximum(occ[:, 0:1] * w1_ref[0:1, :] + occ[:, 1:2] * w1_ref[1:2, :]
                    + b1_ref[0:1, :], 0.0)
    g = jnp.maximum(jnp.dot(g, w2_ref[...], preferred_element_type=_f32)
                    + b2_ref[0:1, :], 0.0)
    g = jnp.maximum(jnp.dot(g, w3_ref[...], preferred_element_type=_f32)
                    + b3_ref[0:1, :], 0.0)
    g = jnp.maximum(jnp.dot(g, w4_ref[...], preferred_element_type=_f32)
                    + b4_ref[0:1, :], 0.0)
    o_ref[0] = jnp.max(g, axis=0, keepdims=True)


def _obj_stage(oc, Ws1, bs1, Ws2, bs2, Ws3, bs3, Ws4, bs4):
    q = oc.shape[1]
    return pl.pallas_call(
        _obj_kernel,
        grid=(_NUM,),
        in_specs=[
            pl.BlockSpec((1, q, 2), lambda n: (n, 0, 0)),
            pl.BlockSpec((2, 64), lambda n: (0, 0)),
            pl.BlockSpec((1, 64), lambda n: (0, 0)),
            pl.BlockSpec((64, 128), lambda n: (0, 0)),
            pl.BlockSpec((1, 128), lambda n: (0, 0)),
            pl.BlockSpec((128, 256), lambda n: (0, 0)),
            pl.BlockSpec((1, 256), lambda n: (0, 0)),
            pl.BlockSpec((256, 512), lambda n: (0, 0)),
            pl.BlockSpec((1, 512), lambda n: (0, 0)),
        ],
        out_specs=pl.BlockSpec((1, 1, 512), lambda n: (n, 0, 0)),
        out_shape=jax.ShapeDtypeStruct((_NUM, 1, 512), _f32),
    )(oc, Ws1, bs1.reshape(1, -1), Ws2, bs2.reshape(1, -1),
      Ws3, bs3.reshape(1, -1), Ws4, bs4.reshape(1, -1))


def _cls_kernel(l1_ref, l3_ref, obj_ref, pos_ref, ngrid_ref,
                a1_ref, a2_ref, bmat_ref, cmat_ref, bf1_ref,
                w2_ref, b2_ref, w3_ref, b3_ref, o_ref):
    rel = pos_ref[0] - ngrid_ref[...]
    y = jnp.dot(l1_ref[0], a1_ref[...], preferred_element_type=_f32)
    y = y + jnp.dot(l3_ref[0], a2_ref[...], preferred_element_type=_f32)
    y = y + jnp.dot(obj_ref[0], bmat_ref[...], preferred_element_type=_f32)
    y = y + rel[:, 0:1] * cmat_ref[0:1, :] + rel[:, 1:2] * cmat_ref[1:2, :]
    y = jnp.maximum(y + bf1_ref[0:1, :], 0.0)
    y = jnp.maximum(jnp.dot(y, w2_ref[...], preferred_element_type=_f32)
                    + b2_ref[0:1, :], 0.0)
    o_ref[0] = jnp.dot(y, w3_ref[...], preferred_element_type=_f32) \
        + b3_ref[0:1, :]


def _cls_stage(l1, l3, obj, pos, ngrid, Wf1, bf1, Wf2, bf2, Wf3, bf3):
    return pl.pallas_call(
        _cls_kernel,
        grid=(_NUM,),
        in_specs=[
            pl.BlockSpec((1, 100, 512), lambda n: (n, 0, 0)),
            pl.BlockSpec((1, 100, 512), lambda n: (n, 0, 0)),
            pl.BlockSpec((1, 1, 512), lambda n: (n, 0, 0)),
            pl.BlockSpec((1, 1, 2), lambda n: (n, 0, 0)),
            pl.BlockSpec((100, 2), lambda n: (0, 0)),
            pl.BlockSpec((512, 1024), lambda n: (0, 0)),
            pl.BlockSpec((512, 1024), lambda n: (0, 0)),
            pl.BlockSpec((512, 1024), lambda n: (0, 0)),
            pl.BlockSpec((2, 1024), lambda n: (0, 0)),
            pl.BlockSpec((1, 1024), lambda n: (0, 0)),
            pl.BlockSpec((1024, 256), lambda n: (0, 0)),
            pl.BlockSpec((1, 256), lambda n: (0, 0)),
            pl.BlockSpec((256, 1), lambda n: (0, 0)),
            pl.BlockSpec((1, 1), lambda n: (0, 0)),
        ],
        out_specs=pl.BlockSpec((1, 100, 1), lambda n: (n, 0, 0)),
        out_shape=jax.ShapeDtypeStruct((_NUM, 100, 1), _f32),
    )(l1, l3, obj, pos.reshape(_NUM, 1, 2), ngrid,
      Wf1[:512], Wf1[512:1024], Wf1[1024:1536], Wf1[1536:1538],
      bf1.reshape(1, -1), Wf2, bf2.reshape(1, -1), Wf3, bf3.reshape(1, -1))


def kernel(sc, oc, pos, device, W1, b1, W2, b2, Wc0, bc0, Wc1, bc1, Wt, bt,
           Ws1, bs1, Ws2, bs2, Ws3, bs3, Ws4, bs4,
           Wf1, bf1, Wf2, bf2, Wf3, bf3):
    cut = jnp.linspace(-0.5, 0.5, _CS + 1, dtype=_f32).reshape(1, _CS + 1)
    ncut = jnp.linspace(-0.5, 0.5, 11, dtype=_f32)
    nctr = (ncut[:-1] + ncut[1:]) * 0.5
    ngrid = jnp.stack(jnp.meshgrid(nctr, nctr, indexing='ij'), axis=-1)
    ngrid = ngrid.reshape(100, 2)

    h, seg = _point_stage(sc, cut, W1, b1, W2, b2)
    vox = _scatter_stage(h, seg.reshape(_NUM, _P))

    wc0 = Wc0.transpose(2, 3, 1, 0).reshape(9 * _F, 512)
    l1 = _conv0_stage(vox, wc0, bc0)
    wc1 = Wc1.transpose(2, 3, 1, 0).reshape(9 * 512, 1024)
    l2 = _conv1_stage(l1, wc1, bc1)
    wt = Wt.transpose(0, 2, 3, 1).reshape(1024, 4 * 512)
    l3 = _deconv_stage(l2, wt, bt)

    obj = _obj_stage(oc, Ws1, bs1, Ws2, bs2, Ws3, bs3, Ws4, bs4)
    y = _cls_stage(l1, l3, obj, pos, ngrid, Wf1, bf1, Wf2, bf2, Wf3, bf3)
    return y


# speedup vs baseline: 12.5187x; 1.0981x over previous
import functools

import jax
import jax.numpy as jnp
from jax import lax
from jax.experimental import pallas as pl
from jax.experimental.pallas import tpu as pltpu
from jax.experimental.pallas import tpu_sc as plsc

_NUM = 16
_P = 2048
_CS = 20
_NV = _CS * _CS
_F = 256
_HALF = 128
_CH = 256

_f32 = jnp.float32


def _point_kernel(sc_ref, cut_ref, w1_ref, b1_ref, w2_ref, b2_ref,
                  h_ref, seg_ref):
    v = sc_ref[0]
    x = v[:, 0:1]
    y = v[:, 1:2]
    cut = cut_ref[0:1, :]
    centers = (cut[:, 0:_CS] + cut[:, 1:_CS + 1]) * 0.5

    ix = jnp.sum((x > cut).astype(jnp.int32), axis=1, keepdims=True) - 1
    iy = jnp.sum((y > cut).astype(jnp.int32), axis=1, keepdims=True) - 1
    valid = (ix >= 0) & (ix < _CS) & (iy >= 0) & (iy < _CS)
    ixc = jnp.clip(ix, 0, _CS - 1)
    iyc = jnp.clip(iy, 0, _CS - 1)

    lane = lax.broadcasted_iota(jnp.int32, (_P, _CS), 1)
    ctrx = jnp.sum(jnp.where(lane == ixc, centers, 0.0), axis=1, keepdims=True)
    ctry = jnp.sum(jnp.where(lane == iyc, centers, 0.0), axis=1, keepdims=True)

    px = x - ctrx
    py = y - ctry
    h1 = jnp.maximum(px * w1_ref[0:1, :] + py * w1_ref[1:2, :] + b1_ref[0:1, :], 0.0)
    h = jnp.dot(h1, w2_ref[...], preferred_element_type=_f32) + b2_ref[0:1, :]
    h_ref[0] = jnp.maximum(h, 0.0)
    seg_ref[0] = jnp.where(valid, ixc * _CS + iyc, _NV)


def _point_stage(sc, cut, W1, b1, W2, b2):
    return pl.pallas_call(
        _point_kernel,
        grid=(_NUM,),
        in_specs=[
            pl.BlockSpec((1, _P, 2), lambda n: (n, 0, 0)),
            pl.BlockSpec((1, _CS + 1), lambda n: (0, 0)),
            pl.BlockSpec((2, 128), lambda n: (0, 0)),
            pl.BlockSpec((1, 128), lambda n: (0, 0)),
            pl.BlockSpec((128, _F), lambda n: (0, 0)),
            pl.BlockSpec((1, _F), lambda n: (0, 0)),
        ],
        out_specs=[
            pl.BlockSpec((1, _P, _F), lambda n: (n, 0, 0)),
            pl.BlockSpec((1, _P, 1), lambda n: (n, 0, 0)),
        ],
        out_shape=[
            jax.ShapeDtypeStruct((_NUM, _P, _F), _f32),
            jax.ShapeDtypeStruct((_NUM, _P, 1), jnp.int32),
        ],
    )(sc, cut, W1, b1.reshape(1, -1), W2, b2.reshape(1, -1))


def _scatter_max_body(h_hbm, seg_hbm, vox_hbm, acc, hb0, hb1, segbuf,
                      sem0, sem1):
    scene = lax.axis_index("s")
    half = lax.axis_index("c")

    @pl.loop(0, _NV + 1)
    def _zero(r):
        for j in range(_HALF // 16):
            acc[r, pl.ds(j * 16, 16)] = jnp.zeros((16,), _f32)

    pltpu.sync_copy(seg_hbm.at[scene], segbuf.at[pl.ds(0, _P)])

    bufs = (hb0, hb1)
    sems = (sem0, sem1)
    nchunk = _P // _CH

    def _start(c):
        return pltpu.async_copy(
            h_hbm.at[scene, pl.ds(c * _CH, _CH), pl.ds(half * _HALF, _HALF)],
            bufs[c % 2], sems[c % 2])

    pending = {0: _start(0)}
    for c in range(nchunk):
        if c + 1 < nchunk:
            pending[c + 1] = _start(c + 1)
        pending[c].wait()
        hbuf = bufs[c % 2]

        @pl.loop(0, _CH, step=16)
        def _grp(p0, c=c, hbuf=hbuf):
            sv = segbuf[pl.ds(c * _CH + p0, 16)]
            for k in range(16):
                s = sv[k]

                @pl.when(s < _NV)
                def _upd(s=s, k=k):
                    for j in range(_HALF // 16):
                        sl = pl.ds(j * 16, 16)
                        acc[s, sl] = jnp.maximum(acc[s, sl], hbuf[p0 + k, sl])

    pltpu.sync_copy(acc.at[pl.ds(0, _NV), :],
                    vox_hbm.at[scene, :, pl.ds(half * _HALF, _HALF)])


def _scatter_stage(h, seg):
    mesh = plsc.VectorSubcoreMesh(core_axis_name="c", subcore_axis_name="s")
    run = pl.kernel(
        _scatter_max_body,
        out_type=jax.ShapeDtypeStruct((_NUM, _NV, _F), _f32),
        mesh=mesh,
        scratch_types=[
            pltpu.VMEM((_NV + 1, _HALF), _f32),
            pltpu.VMEM((_CH, _HALF), _f32),
            pltpu.VMEM((_CH, _HALF), _f32),
            pltpu.VMEM((_P + 16,), jnp.int32),
            pltpu.SemaphoreType.DMA,
            pltpu.SemaphoreType.DMA,
        ],
    )
    return run(h, seg)


def _conv_pool_kernel(hw, hwp, rows, x_ref, w_ref, b_ref, o_ref, pad_ref):
    c_in = x_ref.shape[2]
    c_out = w_ref.shape[1]
    pad_ref[...] = jnp.zeros_like(pad_ref)
    xv = x_ref[0]
    for r in range(hw):
        pad_ref[hwp * (r + 1) + 1:hwp * (r + 1) + 1 + hw, :] = \
            xv[hw * r:hw * r + hw, :]
    acc = jnp.zeros((rows, c_out), _f32)
    for d in range(9):
        off = hwp * (d // 3) + (d % 3)
        acc = acc + jnp.dot(pad_ref[off:off + rows, :],
                            w_ref[d * c_in:(d + 1) * c_in, :],
                            preferred_element_type=_f32)
    act = jnp.maximum(acc + b_ref[...].reshape(1, c_out), 0.0)
    gr = act[:hw * hwp].reshape(hw, hwp, c_out)[:, :hw, :]
    p = gr.reshape(hw // 2, 2, hw // 2, 2, c_out)
    pooled = jnp.max(jnp.max(p, axis=3), axis=1)
    o_ref[0] = pooled.reshape((hw // 2) * (hw // 2), c_out)


def _conv0_stage(vox, w, b):
    kfn = functools.partial(_conv_pool_kernel, 20, 22, 448)
    return pl.pallas_call(
        kfn,
        grid=(_NUM,),
        in_specs=[
            pl.BlockSpec((1, _NV, _F), lambda n: (n, 0, 0)),
            pl.BlockSpec((9 * _F, 512), lambda n: (0, 0)),
            pl.BlockSpec((1, 512), lambda n: (0, 0)),
        ],
        out_specs=pl.BlockSpec((1, 100, 512), lambda n: (n, 0, 0)),
        out_shape=jax.ShapeDtypeStruct((_NUM, 100, 512), _f32),
        scratch_shapes=[pltpu.VMEM((22 * 22 + 60, _F), _f32)],
    )(vox, w, b.reshape(1, -1))


def _conv1_stage(l1, w, b):
    kfn = functools.partial(_conv_pool_kernel, 10, 12, 128)
    return pl.pallas_call(
        kfn,
        grid=(_NUM, 2),
        in_specs=[
            pl.BlockSpec((1, 100, 512), lambda n, o: (n, 0, 0)),
            pl.BlockSpec((9 * 512, 512), lambda n, o: (0, o)),
            pl.BlockSpec((1, 1, 512), lambda n, o: (o, 0, 0)),
        ],
        out_specs=pl.BlockSpec((1, 25, 512), lambda n, o: (n, 0, o)),
        out_shape=jax.ShapeDtypeStruct((_NUM, 25, 1024), _f32),
        scratch_shapes=[pltpu.VMEM((12 * 12 + 40, 512), _f32)],
    )(l1, w, b.reshape(2, 1, 512))


def _deconv_kernel(x_ref, w_ref, b_ref, o_ref):
    t = jnp.dot(x_ref[0], w_ref[...], preferred_element_type=_f32)
    t = t.reshape(5, 5, 2, 2, 512)
    t = jnp.transpose(t, (0, 2, 1, 3, 4))
    o_ref[0] = t.reshape(100, 512) + b_ref[0:1, :]


def _deconv_stage(l2, wt, bt):
    return pl.pallas_call(
        _deconv_kernel,
        grid=(_NUM,),
        in_specs=[
            pl.BlockSpec((1, 25, 1024), lambda n: (n, 0, 0)),
            pl.BlockSpec((1024, 2048), lambda n: (0, 0)),
            pl.BlockSpec((1, 512), lambda n: (0, 0)),
        ],
        out_specs=pl.BlockSpec((1, 100, 512), lambda n: (n, 0, 0)),
        out_shape=jax.ShapeDtypeStruct((_NUM, 100, 512), _f32),
    )(l2, wt, bt.reshape(1, -1))


def _obj_kernel(oc_ref, w1_ref, b1_ref, w2_ref, b2_ref, w3_ref, b3_ref,
                w4_ref, b4_ref, o_ref):
    v = oc_ref[0]
    occ = v - jnp.mean(v, axis=0, keepdims=True)
    g = jnp.maximum(occ[:, 0:1] * w1_ref[0:1, :] + occ[:, 1:2] * w1_ref[1:2, :]
                    + b1_ref[0:1, :], 0.0)
    g = jnp.maximum(jnp.dot(g, w2_ref[...], preferred_element_type=_f32)
                    + b2_ref[0:1, :], 0.0)
    g = jnp.maximum(jnp.dot(g, w3_ref[...], preferred_element_type=_f32)
                    + b3_ref[0:1, :], 0.0)
    g = jnp.maximum(jnp.dot(g, w4_ref[...], preferred_element_type=_f32)
                    + b4_ref[0:1, :], 0.0)
    o_ref[0] = jnp.max(g, axis=0, keepdims=True)


def _obj_stage(oc, Ws1, bs1, Ws2, bs2, Ws3, bs3, Ws4, bs4):
    q = oc.shape[1]
    return pl.pallas_call(
        _obj_kernel,
        grid=(_NUM,),
        in_specs=[
            pl.BlockSpec((1, q, 2), lambda n: (n, 0, 0)),
            pl.BlockSpec((2, 64), lambda n: (0, 0)),
            pl.BlockSpec((1, 64), lambda n: (0, 0)),
            pl.BlockSpec((64, 128), lambda n: (0, 0)),
            pl.BlockSpec((1, 128), lambda n: (0, 0)),
            pl.BlockSpec((128, 256), lambda n: (0, 0)),
            pl.BlockSpec((1, 256), lambda n: (0, 0)),
            pl.BlockSpec((256, 512), lambda n: (0, 0)),
            pl.BlockSpec((1, 512), lambda n: (0, 0)),
        ],
        out_specs=pl.BlockSpec((1, 1, 512), lambda n: (n, 0, 0)),
        out_shape=jax.ShapeDtypeStruct((_NUM, 1, 512), _f32),
    )(oc, Ws1, bs1.reshape(1, -1), Ws2, bs2.reshape(1, -1),
      Ws3, bs3.reshape(1, -1), Ws4, bs4.reshape(1, -1))


def _cls_kernel(l1_ref, l3_ref, obj_ref, pos_ref, ngrid_ref,
                a1_ref, a2_ref, bmat_ref, cmat_ref, bf1_ref,
                w2_ref, b2_ref, w3_ref, b3_ref, o_ref):
    rel = pos_ref[0] - ngrid_ref[...]
    y = jnp.dot(l1_ref[0], a1_ref[...], preferred_element_type=_f32)
    y = y + jnp.dot(l3_ref[0], a2_ref[...], preferred_element_type=_f32)
    y = y + jnp.dot(obj_ref[0], bmat_ref[...], preferred_element_type=_f32)
    y = y + rel[:, 0:1] * cmat_ref[0:1, :] + rel[:, 1:2] * cmat_ref[1:2, :]
    y = jnp.maximum(y + bf1_ref[0:1, :], 0.0)
    y = jnp.maximum(jnp.dot(y, w2_ref[...], preferred_element_type=_f32)
                    + b2_ref[0:1, :], 0.0)
    o_ref[0] = jnp.dot(y, w3_ref[...], preferred_element_type=_f32) \
        + b3_ref[0:1, :]


def _cls_stage(l1, l3, obj, pos, ngrid, Wf1, bf1, Wf2, bf2, Wf3, bf3):
    return pl.pallas_call(
        _cls_kernel,
        grid=(_NUM,),
        in_specs=[
            pl.BlockSpec((1, 100, 512), lambda n: (n, 0, 0)),
            pl.BlockSpec((1, 100, 512), lambda n: (n, 0, 0)),
            pl.BlockSpec((1, 1, 512), lambda n: (n, 0, 0)),
            pl.BlockSpec((1, 1, 2), lambda n: (n, 0, 0)),
            pl.BlockSpec((100, 2), lambda n: (0, 0)),
            pl.BlockSpec((512, 1024), lambda n: (0, 0)),
            pl.BlockSpec((512, 1024), lambda n: (0, 0)),
            pl.BlockSpec((512, 1024), lambda n: (0, 0)),
            pl.BlockSpec((2, 1024), lambda n: (0, 0)),
            pl.BlockSpec((1, 1024), lambda n: (0, 0)),
            pl.BlockSpec((1024, 256), lambda n: (0, 0)),
            pl.BlockSpec((1, 256), lambda n: (0, 0)),
            pl.BlockSpec((256, 1), lambda n: (0, 0)),
            pl.BlockSpec((1, 1), lambda n: (0, 0)),
        ],
        out_specs=pl.BlockSpec((1, 100, 1), lambda n: (n, 0, 0)),
        out_shape=jax.ShapeDtypeStruct((_NUM, 100, 1), _f32),
    )(l1, l3, obj, pos.reshape(_NUM, 1, 2), ngrid,
      Wf1[:512], Wf1[512:1024], Wf1[1024:1536], Wf1[1536:1538],
      bf1.reshape(1, -1), Wf2, bf2.reshape(1, -1), Wf3, bf3.reshape(1, -1))


def kernel(sc, oc, pos, device, W1, b1, W2, b2, Wc0, bc0, Wc1, bc1, Wt, bt,
           Ws1, bs1, Ws2, bs2, Ws3, bs3, Ws4, bs4,
           Wf1, bf1, Wf2, bf2, Wf3, bf3):
    cut = jnp.linspace(-0.5, 0.5, _CS + 1, dtype=_f32).reshape(1, _CS + 1)
    ncut = jnp.linspace(-0.5, 0.5, 11, dtype=_f32)
    nctr = (ncut[:-1] + ncut[1:]) * 0.5
    ngrid = jnp.stack(jnp.meshgrid(nctr, nctr, indexing='ij'), axis=-1)
    ngrid = ngrid.reshape(100, 2)

    h, seg = _point_stage(sc, cut, W1, b1, W2, b2)
    vox = _scatter_stage(h, seg.reshape(_NUM, _P))

    wc0 = Wc0.transpose(2, 3, 1, 0).reshape(9 * _F, 512)
    l1 = _conv0_stage(vox, wc0, bc0)
    wc1 = Wc1.transpose(2, 3, 1, 0).reshape(9 * 512, 1024)
    l2 = _conv1_stage(l1, wc1, bc1)
    wt = Wt.transpose(0, 2, 3, 1).reshape(1024, 4 * 512)
    l3 = _deconv_stage(l2, wt, bt)

    obj = _obj_stage(oc, Ws1, bs1, Ws2, bs2, Ws3, bs3, Ws4, bs4)
    y = _cls_stage(l1, l3, obj, pos, ngrid, Wf1, bf1, Wf2, bf2, Wf3, bf3)
    return y
